# baseline (speedup 1.0000x reference)
import functools

import jax
import jax.numpy as jnp
from jax import lax
from jax.experimental import pallas as pl
from jax.experimental.pallas import tpu as pltpu
from jax.experimental.pallas import tpu_sc as plsc

N = 10000
E = 320000
D = 128
R = 8

NC = 2
NS = 16
NW = NC * NS
EPW = E // NW
EPS = E // NS
K = 80
EH1 = 4800
EH2 = 5200
CT_ROWS = 640
RPT = N // NS

_mesh = plsc.VectorSubcoreMesh(
    core_axis_name="c", subcore_axis_name="s", num_cores=NC, num_subcores=NS
)


def _counts_body(dst_hbm, typ_hbm, esc_hbm,
                 cnt_tile, dstb, typb, escb, rowidx, cnt_sh):
  cid = lax.axis_index("c")
  sid = lax.axis_index("s")
  wid = sid * NC + cid

  zi = jnp.zeros((16,), jnp.int32)
  iot = lax.iota(jnp.int32, 16)

  def zero_cnt(i, carry):
    for c in range(8):
      cnt_tile[i, pl.ds(c * 16, 16)] = zi
    return carry
  lax.fori_loop(0, CT_ROWS, zero_cnt, 0)

  for j in range(5):
    for c in range(8):
      rowidx[j, pl.ds(c * 16, 16)] = j * 128 + c * 16 + iot

  @pl.when(sid == 0)
  def _():
    pltpu.sync_copy(cnt_tile, cnt_sh)
  plsc.subcore_barrier()

  ones = jnp.ones((16,), jnp.int32)
  for h in range(EPS // EPW):
    off = sid * EPS + ((1 - cid) if h == 0 else cid) * EPW
    pltpu.sync_copy(dst_hbm.at[pl.ds(off, EPW)], dstb)
    pltpu.sync_copy(typ_hbm.at[pl.ds(off, EPW)], typb)

    def cnt_step(i, carry):
      d16 = dstb[pl.ds(i * 16, 16)]
      t16 = typb[pl.ds(i * 16, 16)]
      k16 = d16 * R + t16
      plsc.addupdate_scatter(cnt_tile, [k16 >> 7, k16 & 127], ones)
      return carry
    lax.fori_loop(0, EPW // 16, cnt_step, 0, unroll=4)

  for j in range(5):
    pltpu.sync_copy(cnt_tile.at[pl.ds(j * 128, 128)],
                    cnt_sh.at[rowidx.at[j]], add=True)
  plsc.subcore_barrier()
  pltpu.sync_copy(cnt_sh, cnt_tile)

  def esc_step(i, carry):
    d16 = dstb[pl.ds(i * 16, 16)]
    t16 = typb[pl.ds(i * 16, 16)]
    k16 = d16 * R + t16
    c16 = plsc.load_gather(cnt_tile, [k16 >> 7, k16 & 127])
    escb[pl.ds(i * 16, 16)] = 1.0 / c16.astype(jnp.float32)
    return carry
  lax.fori_loop(0, EPW // 16, esc_step, 0, unroll=4)
  pltpu.sync_copy(escb, esc_hbm.at[pl.ds(wid * EPW, EPW)])


_sc_counts = functools.partial(
    pl.kernel,
    out_type=jax.ShapeDtypeStruct((E,), jnp.float32),
    mesh=_mesh,
    scratch_types=[
        pltpu.VMEM((CT_ROWS, 128), jnp.int32),
        pltpu.VMEM((EPW,), jnp.int32),
        pltpu.VMEM((EPW,), jnp.int32),
        pltpu.VMEM((EPW,), jnp.float32),
        pltpu.VMEM((5, 128), jnp.int32),
        pltpu.VMEM_SHARED((CT_ROWS, 128), jnp.int32),
    ],
    compiler_params=pltpu.CompilerParams(needs_layout_passes=False),
)(_counts_body)


def _agg_body(src_hbm, dst_hbm, typ_hbm, esc_hbm, xw_hbm, out_hbm,
              srcb, dstb, typb, escb, gidx, sdst, rows, zbuf, agg_sh,
              gsem0, gsem1):
  cid = lax.axis_index("c")
  sid = lax.axis_index("s")
  wid = sid * NC + cid
  gsem = (gsem0, gsem1)

  zf = jnp.zeros((16,), jnp.float32)

  def zero_z(i, carry):
    for c in range(8):
      zbuf[i, pl.ds(c * 16, 16)] = zf
    return carry
  lax.fori_loop(0, 25, zero_z, 0)
  for z in range(RPT // 25):
    pltpu.async_copy(zbuf, agg_sh.at[pl.ds(sid * RPT + z * 25, 25)], gsem0)
  for z in range(RPT // 25):
    pltpu.make_async_copy(
        zbuf, agg_sh.at[pl.ds(sid * RPT + z * 25, 25)], gsem0).wait()

  plsc.subcore_barrier()

  def start_gather(slot):
    pltpu.async_copy(xw_hbm.at[gidx.at[slot]], rows.at[slot], gsem[slot])

  def wait_gather(slot):
    pltpu.make_async_copy(xw_hbm.at[gidx.at[slot]], rows.at[slot],
                          gsem[slot]).wait()

  def scale(base, slot):
    for g in range(K // 16):
      e16 = escb[pl.ds(base + g * 16, 16)]
      for e in range(16):
        r = g * 16 + e
        b = e16[e]
        for c in range(8):
          rows[slot, r, pl.ds(c * 16, 16)] = (
              rows[slot, r, pl.ds(c * 16, 16)] * b)

  def scatter(slot):
    pltpu.sync_copy(rows.at[slot], agg_sh.at[sdst.at[slot]], add=True)

  for off, sz in ((0, EH1), (EH1, EH2)):
    nch = sz // K
    gbase = wid * EPW + off
    pltpu.sync_copy(src_hbm.at[pl.ds(gbase, sz)], srcb.at[pl.ds(0, sz)])
    pltpu.sync_copy(dst_hbm.at[pl.ds(gbase, sz)], dstb.at[pl.ds(0, sz)])
    pltpu.sync_copy(typ_hbm.at[pl.ds(gbase, sz)], typb.at[pl.ds(0, sz)])
    pltpu.sync_copy(esc_hbm.at[pl.ds(gbase, sz)], escb.at[pl.ds(0, sz)])

    def build_idx(base, slot):
      for g in range(K // 16):
        s16 = srcb[pl.ds(base + g * 16, 16)]
        t16 = typb[pl.ds(base + g * 16, 16)]
        gidx[slot, pl.ds(g * 16, 16)] = t16 * N + s16
        sdst[slot, pl.ds(g * 16, 16)] = dstb[pl.ds(base + g * 16, 16)]

    build_idx(0, 0)
    start_gather(0)

    def pair(m, carry):
      b1 = 2 * m * K + K
      b2 = jnp.minimum(b1 + K, sz - K)
      build_idx(b1, 1)
      start_gather(1)
      wait_gather(0)
      scale(b1 - K, 0)
      scatter(0)
      build_idx(b2, 0)
      start_gather(0)
      wait_gather(1)
      scale(b1, 1)
      scatter(1)
      return carry
    lax.fori_loop(0, nch // 2, pair, 0)

    wait_gather(0)
    if nch % 2:
      scale(sz - K, 0)
      scatter(0)

  plsc.subcore_barrier()

  @pl.when(sid < NS - 1)
  def _():
    pltpu.sync_copy(agg_sh.at[pl.ds(sid * 632, 632)],
                    out_hbm.at[cid, pl.ds(sid * 632, 632)])

  @pl.when(sid == NS - 1)
  def _():
    pltpu.sync_copy(agg_sh.at[pl.ds((NS - 1) * 632, N - (NS - 1) * 632)],
                    out_hbm.at[cid, pl.ds((NS - 1) * 632, N - (NS - 1) * 632)])


_sc_agg = functools.partial(
    pl.kernel,
    out_type=jax.ShapeDtypeStruct((NC, N, D), jnp.float32),
    mesh=_mesh,
    scratch_types=[
        pltpu.VMEM((EH2,), jnp.int32),
        pltpu.VMEM((EH2,), jnp.int32),
        pltpu.VMEM((EH2,), jnp.int32),
        pltpu.VMEM((EH2,), jnp.float32),
        pltpu.VMEM((2, K), jnp.int32),
        pltpu.VMEM((2, K), jnp.int32),
        pltpu.VMEM((2, K, D), jnp.float32),
        pltpu.VMEM((25, D), jnp.float32),
        pltpu.VMEM_SHARED((N, D), jnp.float32),
        pltpu.SemaphoreType.DMA,
        pltpu.SemaphoreType.DMA,
    ],
    compiler_params=pltpu.CompilerParams(needs_layout_passes=False),
)(_agg_body)


def _xw_body(x_ref, w_ref, o_ref):
  o_ref[0] = jnp.dot(x_ref[...], w_ref[0], preferred_element_type=jnp.float32)


def _final_body(x_ref, root_ref, b_ref, p_ref, o_ref):
  o_ref[...] = (
      jnp.dot(x_ref[...], root_ref[...], preferred_element_type=jnp.float32)
      + p_ref[0] + p_ref[1] + b_ref[...]
  )


def kernel(x, edge_index, edge_type, edge_attr, bases, comp, root, bias):
  weight = jnp.einsum("rb,bio->rio", comp, bases)

  esc = _sc_counts(edge_index[1], edge_type)

  xw = pl.pallas_call(
      _xw_body,
      grid=(R,),
      in_specs=[
          pl.BlockSpec((N, D), lambda r: (0, 0)),
          pl.BlockSpec((1, D, D), lambda r: (r, 0, 0)),
      ],
      out_specs=pl.BlockSpec((1, N, D), lambda r: (r, 0, 0)),
      out_shape=jax.ShapeDtypeStruct((R, N, D), jnp.float32),
  )(x, weight)
  xw_flat = xw.reshape(R * N, D)

  src = edge_index[0]
  dst = edge_index[1]

  partial = _sc_agg(src, dst, edge_type, esc, xw_flat)

  bn = 1000
  out = pl.pallas_call(
      _final_body,
      grid=(N // bn,),
      in_specs=[
          pl.BlockSpec((bn, D), lambda i: (i, 0)),
          pl.BlockSpec((D, D), lambda i: (0, 0)),
          pl.BlockSpec((1, D), lambda i: (0, 0)),
          pl.BlockSpec((NC, bn, D), lambda i: (0, i, 0)),
      ],
      out_specs=pl.BlockSpec((bn, D), lambda i: (i, 0)),
      out_shape=jax.ShapeDtypeStruct((N, D), jnp.float32),
  )(x, root, bias.reshape(1, D), partial)

  return (out, edge_attr)


# speedup vs baseline: 1.0108x; 1.0108x over previous
import functools

import jax
import jax.numpy as jnp
from jax import lax
from jax.experimental import pallas as pl
from jax.experimental.pallas import tpu as pltpu
from jax.experimental.pallas import tpu_sc as plsc

N = 10000
E = 320000
D = 128
R = 8

NC = 2
NS = 16
NW = NC * NS
EPW = E // NW
EPS = E // NS
K = 80
EH1 = 4800
EH2 = 5200
CT_ROWS = 640
RPT = N // NS

_mesh = plsc.VectorSubcoreMesh(
    core_axis_name="c", subcore_axis_name="s", num_cores=NC, num_subcores=NS
)


def _counts_body(dst_hbm, typ_hbm, esc_hbm,
                 cnt_tile, dstb, typb, escb, rowidx, cnt_sh):
  cid = lax.axis_index("c")
  sid = lax.axis_index("s")
  wid = sid * NC + cid

  zi = jnp.zeros((16,), jnp.int32)
  iot = lax.iota(jnp.int32, 16)

  def zero_cnt(i, carry):
    for c in range(8):
      cnt_tile[i, pl.ds(c * 16, 16)] = zi
    return carry
  lax.fori_loop(0, CT_ROWS, zero_cnt, 0)

  for j in range(5):
    for c in range(8):
      rowidx[j, pl.ds(c * 16, 16)] = j * 128 + c * 16 + iot

  @pl.when(sid == 0)
  def _():
    pltpu.sync_copy(cnt_tile, cnt_sh)
  plsc.subcore_barrier()

  ones = jnp.ones((16,), jnp.int32)
  for h in range(EPS // EPW):
    off = sid * EPS + ((1 - cid) if h == 0 else cid) * EPW
    pltpu.sync_copy(dst_hbm.at[pl.ds(off, EPW)], dstb)
    pltpu.sync_copy(typ_hbm.at[pl.ds(off, EPW)], typb)

    def cnt_step(i, carry):
      d16 = dstb[pl.ds(i * 16, 16)]
      t16 = typb[pl.ds(i * 16, 16)]
      k16 = d16 * R + t16
      plsc.addupdate_scatter(cnt_tile, [k16 >> 7, k16 & 127], ones)
      return carry
    lax.fori_loop(0, EPW // 16, cnt_step, 0)

  for j in range(5):
    pltpu.sync_copy(cnt_tile.at[pl.ds(j * 128, 128)],
                    cnt_sh.at[rowidx.at[j]], add=True)
  plsc.subcore_barrier()
  pltpu.sync_copy(cnt_sh, cnt_tile)

  def esc_step(i, carry):
    d16 = dstb[pl.ds(i * 16, 16)]
    t16 = typb[pl.ds(i * 16, 16)]
    k16 = d16 * R + t16
    c16 = plsc.load_gather(cnt_tile, [k16 >> 7, k16 & 127])
    escb[pl.ds(i * 16, 16)] = 1.0 / c16.astype(jnp.float32)
    return carry
  lax.fori_loop(0, EPW // 16, esc_step, 0)
  pltpu.sync_copy(escb, esc_hbm.at[pl.ds(wid * EPW, EPW)])


_sc_counts = functools.partial(
    pl.kernel,
    out_type=jax.ShapeDtypeStruct((E,), jnp.float32),
    mesh=_mesh,
    scratch_types=[
        pltpu.VMEM((CT_ROWS, 128), jnp.int32),
        pltpu.VMEM((EPW,), jnp.int32),
        pltpu.VMEM((EPW,), jnp.int32),
        pltpu.VMEM((EPW,), jnp.float32),
        pltpu.VMEM((5, 128), jnp.int32),
        pltpu.VMEM_SHARED((CT_ROWS, 128), jnp.int32),
    ],
    compiler_params=pltpu.CompilerParams(needs_layout_passes=False),
)(_counts_body)


def _agg_body(src_hbm, dst_hbm, typ_hbm, esc_hbm, xw_hbm, out_hbm,
              srcb, dstb, typb, escb, gidx, sdst, rows, zbuf, agg_sh,
              gsem0, gsem1):
  cid = lax.axis_index("c")
  sid = lax.axis_index("s")
  wid = sid * NC + cid
  gsem = (gsem0, gsem1)

  zf = jnp.zeros((16,), jnp.float32)

  def zero_z(i, carry):
    for c in range(8):
      zbuf[i, pl.ds(c * 16, 16)] = zf
    return carry
  lax.fori_loop(0, 25, zero_z, 0)
  for z in range(RPT // 25):
    pltpu.async_copy(zbuf, agg_sh.at[pl.ds(sid * RPT + z * 25, 25)], gsem0)
  for z in range(RPT // 25):
    pltpu.make_async_copy(
        zbuf, agg_sh.at[pl.ds(sid * RPT + z * 25, 25)], gsem0).wait()

  plsc.subcore_barrier()

  def start_gather(slot):
    pltpu.async_copy(xw_hbm.at[gidx.at[slot]], rows.at[slot], gsem[slot])

  def wait_gather(slot):
    pltpu.make_async_copy(xw_hbm.at[gidx.at[slot]], rows.at[slot],
                          gsem[slot]).wait()

  def scale(base, slot):
    for g in range(K // 16):
      e16 = escb[pl.ds(base + g * 16, 16)]
      for e in range(16):
        r = g * 16 + e
        b = e16[e]
        for c in range(8):
          rows[slot, r, pl.ds(c * 16, 16)] = (
              rows[slot, r, pl.ds(c * 16, 16)] * b)

  def scatter(slot):
    pltpu.sync_copy(rows.at[slot], agg_sh.at[sdst.at[slot]], add=True)

  for off, sz in ((0, EH1), (EH1, EH2)):
    nch = sz // K
    gbase = wid * EPW + off
    pltpu.sync_copy(src_hbm.at[pl.ds(gbase, sz)], srcb.at[pl.ds(0, sz)])
    pltpu.sync_copy(dst_hbm.at[pl.ds(gbase, sz)], dstb.at[pl.ds(0, sz)])
    pltpu.sync_copy(typ_hbm.at[pl.ds(gbase, sz)], typb.at[pl.ds(0, sz)])
    pltpu.sync_copy(esc_hbm.at[pl.ds(gbase, sz)], escb.at[pl.ds(0, sz)])

    def build_idx(base, slot):
      for g in range(K // 16):
        s16 = srcb[pl.ds(base + g * 16, 16)]
        t16 = typb[pl.ds(base + g * 16, 16)]
        gidx[slot, pl.ds(g * 16, 16)] = t16 * N + s16
        sdst[slot, pl.ds(g * 16, 16)] = dstb[pl.ds(base + g * 16, 16)]

    build_idx(0, 0)
    start_gather(0)

    def pair(m, carry):
      b1 = 2 * m * K + K
      b2 = jnp.minimum(b1 + K, sz - K)
      build_idx(b1, 1)
      start_gather(1)
      wait_gather(0)
      scale(b1 - K, 0)
      scatter(0)
      build_idx(b2, 0)
      start_gather(0)
      wait_gather(1)
      scale(b1, 1)
      scatter(1)
      return carry
    lax.fori_loop(0, nch // 2, pair, 0)

    wait_gather(0)
    if nch % 2:
      scale(sz - K, 0)
      scatter(0)

  plsc.subcore_barrier()

  @pl.when(sid < NS - 1)
  def _():
    pltpu.sync_copy(agg_sh.at[pl.ds(sid * 632, 632)],
                    out_hbm.at[cid, pl.ds(sid * 632, 632)])

  @pl.when(sid == NS - 1)
  def _():
    pltpu.sync_copy(agg_sh.at[pl.ds((NS - 1) * 632, N - (NS - 1) * 632)],
                    out_hbm.at[cid, pl.ds((NS - 1) * 632, N - (NS - 1) * 632)])


_sc_agg = functools.partial(
    pl.kernel,
    out_type=jax.ShapeDtypeStruct((NC, N, D), jnp.float32),
    mesh=_mesh,
    scratch_types=[
        pltpu.VMEM((EH2,), jnp.int32),
        pltpu.VMEM((EH2,), jnp.int32),
        pltpu.VMEM((EH2,), jnp.int32),
        pltpu.VMEM((EH2,), jnp.float32),
        pltpu.VMEM((2, K), jnp.int32),
        pltpu.VMEM((2, K), jnp.int32),
        pltpu.VMEM((2, K, D), jnp.float32),
        pltpu.VMEM((25, D), jnp.float32),
        pltpu.VMEM_SHARED((N, D), jnp.float32),
        pltpu.SemaphoreType.DMA,
        pltpu.SemaphoreType.DMA,
    ],
    compiler_params=pltpu.CompilerParams(needs_layout_passes=False),
)(_agg_body)


def _xw_body(x_ref, w_ref, o_ref):
  o_ref[0] = jnp.dot(x_ref[...], w_ref[0], preferred_element_type=jnp.float32)


def _final_body(x_ref, root_ref, b_ref, p_ref, o_ref):
  o_ref[...] = (
      jnp.dot(x_ref[...], root_ref[...], preferred_element_type=jnp.float32)
      + p_ref[0] + p_ref[1] + b_ref[...]
  )


def kernel(x, edge_index, edge_type, edge_attr, bases, comp, root, bias):
  weight = jnp.einsum("rb,bio->rio", comp, bases)

  esc = _sc_counts(edge_index[1], edge_type)

  xw = pl.pallas_call(
      _xw_body,
      grid=(R,),
      in_specs=[
          pl.BlockSpec((N, D), lambda r: (0, 0)),
          pl.BlockSpec((1, D, D), lambda r: (r, 0, 0)),
      ],
      out_specs=pl.BlockSpec((1, N, D), lambda r: (r, 0, 0)),
      out_shape=jax.ShapeDtypeStruct((R, N, D), jnp.float32),
  )(x, weight)
  xw_flat = xw.reshape(R * N, D)

  src = edge_index[0]
  dst = edge_index[1]

  partial = _sc_agg(src, dst, edge_type, esc, xw_flat)

  bn = 1000
  out = pl.pallas_call(
      _final_body,
      grid=(N // bn,),
      in_specs=[
          pl.BlockSpec((bn, D), lambda i: (i, 0)),
          pl.BlockSpec((D, D), lambda i: (0, 0)),
          pl.BlockSpec((1, D), lambda i: (0, 0)),
          pl.BlockSpec((NC, bn, D), lambda i: (0, i, 0)),
      ],
      out_specs=pl.BlockSpec((bn, D), lambda i: (i, 0)),
      out_shape=jax.ShapeDtypeStruct((N, D), jnp.float32),
  )(x, root, bias.reshape(1, D), partial)

  return (out, edge_attr)


# speedup vs baseline: 1.0856x; 1.0740x over previous
import functools

import jax
import jax.numpy as jnp
from jax import lax
from jax.experimental import pallas as pl
from jax.experimental.pallas import tpu as pltpu
from jax.experimental.pallas import tpu_sc as plsc

N = 10000
E = 320000
D = 128
R = 8

NC = 2
NS = 16
NW = NC * NS
EPW = E // NW
EPS = E // NS
K = 80
EH1 = 4800
EH2 = 5200
CT_ROWS = 640
RPT = N // NS

_mesh = plsc.VectorSubcoreMesh(
    core_axis_name="c", subcore_axis_name="s", num_cores=NC, num_subcores=NS
)


def _counts_body(dst_hbm, typ_hbm, esc_hbm,
                 cnt_tile, dstb, typb, escb, rowidx, cnt_sh):
  cid = lax.axis_index("c")
  sid = lax.axis_index("s")
  wid = sid * NC + cid

  zi = jnp.zeros((16,), jnp.int32)
  iot = lax.iota(jnp.int32, 16)

  def zero_cnt(i, carry):
    for c in range(8):
      cnt_tile[i, pl.ds(c * 16, 16)] = zi
    return carry
  lax.fori_loop(0, CT_ROWS, zero_cnt, 0)

  for j in range(5):
    for c in range(8):
      rowidx[j, pl.ds(c * 16, 16)] = j * 128 + c * 16 + iot

  @pl.when(sid == 0)
  def _():
    pltpu.sync_copy(cnt_tile, cnt_sh)
  plsc.subcore_barrier()

  ones = jnp.ones((16,), jnp.int32)
  for h in range(EPS // EPW):
    off = sid * EPS + ((1 - cid) if h == 0 else cid) * EPW
    pltpu.sync_copy(dst_hbm.at[pl.ds(off, EPW)], dstb)
    pltpu.sync_copy(typ_hbm.at[pl.ds(off, EPW)], typb)

    def cnt_step(i, carry):
      d16 = dstb[pl.ds(i * 16, 16)]
      t16 = typb[pl.ds(i * 16, 16)]
      k16 = d16 * R + t16
      plsc.addupdate_scatter(cnt_tile, [k16 >> 7, k16 & 127], ones)
      return carry
    lax.fori_loop(0, EPW // 16, cnt_step, 0)

  for j in range(5):
    pltpu.sync_copy(cnt_tile.at[pl.ds(j * 128, 128)],
                    cnt_sh.at[rowidx.at[j]], add=True)
  plsc.subcore_barrier()
  pltpu.sync_copy(cnt_sh, cnt_tile)

  def esc_step(i, carry):
    d16 = dstb[pl.ds(i * 16, 16)]
    t16 = typb[pl.ds(i * 16, 16)]
    k16 = d16 * R + t16
    c16 = plsc.load_gather(cnt_tile, [k16 >> 7, k16 & 127])
    escb[pl.ds(i * 16, 16)] = 1.0 / c16.astype(jnp.float32)
    return carry
  lax.fori_loop(0, EPW // 16, esc_step, 0)
  pltpu.sync_copy(escb, esc_hbm.at[pl.ds(wid * EPW, EPW)])


_sc_counts = functools.partial(
    pl.kernel,
    out_type=jax.ShapeDtypeStruct((E,), jnp.float32),
    mesh=_mesh,
    scratch_types=[
        pltpu.VMEM((CT_ROWS, 128), jnp.int32),
        pltpu.VMEM((EPW,), jnp.int32),
        pltpu.VMEM((EPW,), jnp.int32),
        pltpu.VMEM((EPW,), jnp.float32),
        pltpu.VMEM((5, 128), jnp.int32),
        pltpu.VMEM_SHARED((CT_ROWS, 128), jnp.int32),
    ],
    compiler_params=pltpu.CompilerParams(needs_layout_passes=False),
)(_counts_body)


def _agg_body(src_hbm, dst_hbm, typ_hbm, esc_hbm, xw_hbm, out_hbm,
              srcb, dstb, typb, escb, gidx, sdst, rows, zbuf, agg_sh,
              gsem0, gsem1):
  cid = lax.axis_index("c")
  sid = lax.axis_index("s")
  wid = sid * NC + cid
  gsem = (gsem0, gsem1)

  zf = jnp.zeros((16,), jnp.float32)

  def zero_z(i, carry):
    for c in range(8):
      zbuf[i, pl.ds(c * 16, 16)] = zf
    return carry
  lax.fori_loop(0, 25, zero_z, 0)
  for z in range(RPT // 25):
    pltpu.sync_copy(zbuf, agg_sh.at[pl.ds(sid * RPT + z * 25, 25)])

  plsc.subcore_barrier()

  def start_gather(slot):
    pltpu.async_copy(xw_hbm.at[gidx.at[slot]], rows.at[slot], gsem[slot])

  def wait_gather(slot):
    pltpu.make_async_copy(xw_hbm.at[gidx.at[slot]], rows.at[slot],
                          gsem[slot]).wait()

  def scale(base, slot):
    for g in range(K // 16):
      e16 = escb[pl.ds(base + g * 16, 16)]
      for e in range(16):
        r = g * 16 + e
        b = e16[e]
        for c in range(8):
          rows[slot, r, pl.ds(c * 16, 16)] = (
              rows[slot, r, pl.ds(c * 16, 16)] * b)

  def scatter(slot):
    pltpu.sync_copy(rows.at[slot], agg_sh.at[sdst.at[slot]], add=True)

  for off, sz in ((0, EH1), (EH1, EH2)):
    nch = sz // K
    gbase = wid * EPW + off
    pltpu.sync_copy(src_hbm.at[pl.ds(gbase, sz)], srcb.at[pl.ds(0, sz)])
    pltpu.sync_copy(dst_hbm.at[pl.ds(gbase, sz)], dstb.at[pl.ds(0, sz)])
    pltpu.sync_copy(typ_hbm.at[pl.ds(gbase, sz)], typb.at[pl.ds(0, sz)])
    pltpu.sync_copy(esc_hbm.at[pl.ds(gbase, sz)], escb.at[pl.ds(0, sz)])

    def build_idx(base, slot):
      for g in range(K // 16):
        s16 = srcb[pl.ds(base + g * 16, 16)]
        t16 = typb[pl.ds(base + g * 16, 16)]
        gidx[slot, pl.ds(g * 16, 16)] = t16 * N + s16
        sdst[slot, pl.ds(g * 16, 16)] = dstb[pl.ds(base + g * 16, 16)]

    build_idx(0, 0)
    start_gather(0)

    def pair(m, carry):
      b1 = 2 * m * K + K
      b2 = jnp.minimum(b1 + K, sz - K)
      build_idx(b1, 1)
      start_gather(1)
      wait_gather(0)
      scale(b1 - K, 0)
      scatter(0)
      build_idx(b2, 0)
      start_gather(0)
      wait_gather(1)
      scale(b1, 1)
      scatter(1)
      return carry
    lax.fori_loop(0, nch // 2, pair, 0)

    wait_gather(0)
    if nch % 2:
      scale(sz - K, 0)
      scatter(0)

  plsc.subcore_barrier()

  @pl.when(sid < NS - 1)
  def _():
    pltpu.sync_copy(agg_sh.at[pl.ds(sid * 632, 632)],
                    out_hbm.at[cid, pl.ds(sid * 632, 632)])

  @pl.when(sid == NS - 1)
  def _():
    pltpu.sync_copy(agg_sh.at[pl.ds((NS - 1) * 632, N - (NS - 1) * 632)],
                    out_hbm.at[cid, pl.ds((NS - 1) * 632, N - (NS - 1) * 632)])


_sc_agg = functools.partial(
    pl.kernel,
    out_type=jax.ShapeDtypeStruct((NC, N, D), jnp.float32),
    mesh=_mesh,
    scratch_types=[
        pltpu.VMEM((EH2,), jnp.int32),
        pltpu.VMEM((EH2,), jnp.int32),
        pltpu.VMEM((EH2,), jnp.int32),
        pltpu.VMEM((EH2,), jnp.float32),
        pltpu.VMEM((2, K), jnp.int32),
        pltpu.VMEM((2, K), jnp.int32),
        pltpu.VMEM((2, K, D), jnp.float32),
        pltpu.VMEM((25, D), jnp.float32),
        pltpu.VMEM_SHARED((N, D), jnp.float32),
        pltpu.SemaphoreType.DMA,
        pltpu.SemaphoreType.DMA,
    ],
    compiler_params=pltpu.CompilerParams(needs_layout_passes=False),
)(_agg_body)


def _xw_body(x_ref, w_ref, o_ref):
  o_ref[0] = jnp.dot(x_ref[...], w_ref[0], preferred_element_type=jnp.float32)


def _final_body(x_ref, root_ref, b_ref, p_ref, o_ref):
  o_ref[...] = (
      jnp.dot(x_ref[...], root_ref[...], preferred_element_type=jnp.float32)
      + p_ref[0] + p_ref[1] + b_ref[...]
  )


def kernel(x, edge_index, edge_type, edge_attr, bases, comp, root, bias):
  weight = jnp.einsum("rb,bio->rio", comp, bases)

  esc = _sc_counts(edge_index[1], edge_type)

  xw = pl.pallas_call(
      _xw_body,
      grid=(R,),
      in_specs=[
          pl.BlockSpec((N, D), lambda r: (0, 0)),
          pl.BlockSpec((1, D, D), lambda r: (r, 0, 0)),
      ],
      out_specs=pl.BlockSpec((1, N, D), lambda r: (r, 0, 0)),
      out_shape=jax.ShapeDtypeStruct((R, N, D), jnp.float32),
  )(x, weight)
  xw_flat = xw.reshape(R * N, D)

  src = edge_index[0]
  dst = edge_index[1]

  partial = _sc_agg(src, dst, edge_type, esc, xw_flat)

  bn = 1000
  out = pl.pallas_call(
      _final_body,
      grid=(N // bn,),
      in_specs=[
          pl.BlockSpec((bn, D), lambda i: (i, 0)),
          pl.BlockSpec((D, D), lambda i: (0, 0)),
          pl.BlockSpec((1, D), lambda i: (0, 0)),
          pl.BlockSpec((NC, bn, D), lambda i: (0, i, 0)),
      ],
      out_specs=pl.BlockSpec((bn, D), lambda i: (i, 0)),
      out_shape=jax.ShapeDtypeStruct((N, D), jnp.float32),
  )(x, root, bias.reshape(1, D), partial)

  return (out, edge_attr)
